# baseline (speedup 1.0000x reference)
import jax
import jax.numpy as jnp
from jax.experimental import pallas as pl
from jax.experimental.pallas import tpu as pltpu

_NBUF = 2


def _make_pipelined_body(n_chunks, tc):
    def _body(x_hbm, w1_ref, b1_ref, w2_ref, b2_ref, out_hbm,
              xbuf, obuf, insem, outsem):
        w1 = w1_ref[...].astype(jnp.bfloat16)
        b1 = b1_ref[...]
        w2 = w2_ref[...].astype(jnp.bfloat16)
        b2 = b2_ref[...]

        def in_copy(i):
            return pltpu.make_async_copy(
                x_hbm.at[pl.ds(i * tc, tc), :],
                xbuf.at[i % _NBUF],
                insem.at[i % _NBUF],
            )

        def out_copy(i):
            return pltpu.make_async_copy(
                obuf.at[i % _NBUF],
                out_hbm.at[pl.ds(i * tc, tc), :],
                outsem.at[i % _NBUF],
            )

        for i in range(min(_NBUF, n_chunks)):
            in_copy(i).start()
        for i in range(n_chunks):
            s = i % _NBUF
            in_copy(i).wait()
            xb = xbuf[s].astype(jnp.bfloat16)
            hid = jnp.dot(xb, w1, preferred_element_type=jnp.float32)
            hid = jnp.maximum(hid + b1, 0.0).astype(jnp.bfloat16)
            if i >= _NBUF:
                out_copy(i - _NBUF).wait()
            obuf[s] = jnp.dot(hid, w2, preferred_element_type=jnp.float32) + b2
            out_copy(i).start()
            if i + _NBUF < n_chunks:
                in_copy(i + _NBUF).start()
        for i in range(max(0, n_chunks - _NBUF), n_chunks):
            out_copy(i).wait()

    return _body


def _emitter_fallback(x, w1, b1, w2, b2):
    def _mlp_body(x_ref, w1_ref, b1_ref, w2_ref, b2_ref, out_ref):
        xb = x_ref[...].astype(jnp.bfloat16)
        w1b = w1_ref[...].astype(jnp.bfloat16)
        hid = jnp.dot(xb, w1b, preferred_element_type=jnp.float32)
        hid = jnp.maximum(hid + b1_ref[...], 0.0).astype(jnp.bfloat16)
        w2b = w2_ref[...].astype(jnp.bfloat16)
        out = jnp.dot(hid, w2b, preferred_element_type=jnp.float32)
        out_ref[...] = out + b2_ref[...]

    B, S = x.shape
    H = w1.shape[1]
    A = w2.shape[1]
    TB = min(8192, B)
    nb = pl.cdiv(B, TB)
    return pl.pallas_call(
        _mlp_body,
        out_shape=jax.ShapeDtypeStruct((B, A), x.dtype),
        grid=(nb,),
        in_specs=[
            pl.BlockSpec((TB, S), lambda i: (i, 0)),
            pl.BlockSpec((S, H), lambda i: (0, 0)),
            pl.BlockSpec((1, H), lambda i: (0, 0)),
            pl.BlockSpec((H, A), lambda i: (0, 0)),
            pl.BlockSpec((1, A), lambda i: (0, 0)),
        ],
        out_specs=pl.BlockSpec((TB, A), lambda i: (i, 0)),
        compiler_params=pltpu.CompilerParams(
            dimension_semantics=("parallel",),
        ),
    )(x, w1, b1, w2, b2)


@jax.jit
def kernel(x, w1, b1, w2, b2):
    B, S = x.shape
    H = w1.shape[1]
    A = w2.shape[1]

    TC = 4096
    if B % TC != 0 or B // TC < 2:
        return _emitter_fallback(x, w1, b1, w2, b2)
    n_chunks = B // TC

    return pl.pallas_call(
        _make_pipelined_body(n_chunks, TC),
        out_shape=jax.ShapeDtypeStruct((B, A), x.dtype),
        in_specs=[
            pl.BlockSpec(memory_space=pltpu.HBM),
            pl.BlockSpec(memory_space=pltpu.VMEM),
            pl.BlockSpec(memory_space=pltpu.VMEM),
            pl.BlockSpec(memory_space=pltpu.VMEM),
            pl.BlockSpec(memory_space=pltpu.VMEM),
        ],
        out_specs=pl.BlockSpec(memory_space=pltpu.HBM),
        scratch_shapes=[
            pltpu.VMEM((_NBUF, TC, S), jnp.float32),
            pltpu.VMEM((_NBUF, TC, A), jnp.float32),
            pltpu.SemaphoreType.DMA((_NBUF,)),
            pltpu.SemaphoreType.DMA((_NBUF,)),
        ],
    )(x, w1, b1, w2, b2)


# speedup vs baseline: 1.0572x; 1.0572x over previous
import jax
import jax.numpy as jnp
from jax.experimental import pallas as pl
from jax.experimental.pallas import tpu as pltpu

_NBUF = 3


def _make_pipelined_body(n_chunks, tc):
    def _body(x_hbm, w1_ref, b1_ref, w2_ref, b2_ref, out_hbm,
              xbuf, obuf, insem, outsem):
        w1 = w1_ref[...].astype(jnp.bfloat16)
        b1 = b1_ref[...]
        w2 = w2_ref[...].astype(jnp.bfloat16)
        b2 = b2_ref[...]

        def in_copy(i):
            return pltpu.make_async_copy(
                x_hbm.at[pl.ds(i * tc, tc), :],
                xbuf.at[i % _NBUF],
                insem.at[i % _NBUF],
            )

        def out_copy(i):
            return pltpu.make_async_copy(
                obuf.at[i % _NBUF],
                out_hbm.at[pl.ds(i * tc, tc), :],
                outsem.at[i % _NBUF],
            )

        for i in range(min(_NBUF, n_chunks)):
            in_copy(i).start()
        for i in range(n_chunks):
            s = i % _NBUF
            in_copy(i).wait()
            xb = xbuf[s].astype(jnp.bfloat16)
            hid = jnp.dot(xb, w1, preferred_element_type=jnp.float32)
            hid = jnp.maximum(hid + b1, 0.0).astype(jnp.bfloat16)
            if i >= _NBUF:
                out_copy(i - _NBUF).wait()
            obuf[s] = jnp.dot(hid, w2, preferred_element_type=jnp.float32) + b2
            out_copy(i).start()
            if i + _NBUF < n_chunks:
                in_copy(i + _NBUF).start()
        for i in range(max(0, n_chunks - _NBUF), n_chunks):
            out_copy(i).wait()

    return _body


def _emitter_fallback(x, w1, b1, w2, b2):
    def _mlp_body(x_ref, w1_ref, b1_ref, w2_ref, b2_ref, out_ref):
        xb = x_ref[...].astype(jnp.bfloat16)
        w1b = w1_ref[...].astype(jnp.bfloat16)
        hid = jnp.dot(xb, w1b, preferred_element_type=jnp.float32)
        hid = jnp.maximum(hid + b1_ref[...], 0.0).astype(jnp.bfloat16)
        w2b = w2_ref[...].astype(jnp.bfloat16)
        out = jnp.dot(hid, w2b, preferred_element_type=jnp.float32)
        out_ref[...] = out + b2_ref[...]

    B, S = x.shape
    H = w1.shape[1]
    A = w2.shape[1]
    TB = min(8192, B)
    nb = pl.cdiv(B, TB)
    return pl.pallas_call(
        _mlp_body,
        out_shape=jax.ShapeDtypeStruct((B, A), x.dtype),
        grid=(nb,),
        in_specs=[
            pl.BlockSpec((TB, S), lambda i: (i, 0)),
            pl.BlockSpec((S, H), lambda i: (0, 0)),
            pl.BlockSpec((1, H), lambda i: (0, 0)),
            pl.BlockSpec((H, A), lambda i: (0, 0)),
            pl.BlockSpec((1, A), lambda i: (0, 0)),
        ],
        out_specs=pl.BlockSpec((TB, A), lambda i: (i, 0)),
        compiler_params=pltpu.CompilerParams(
            dimension_semantics=("parallel",),
        ),
    )(x, w1, b1, w2, b2)


@jax.jit
def kernel(x, w1, b1, w2, b2):
    B, S = x.shape
    H = w1.shape[1]
    A = w2.shape[1]

    TC = 4096
    if B % TC != 0 or B // TC < 2:
        return _emitter_fallback(x, w1, b1, w2, b2)
    n_chunks = B // TC

    return pl.pallas_call(
        _make_pipelined_body(n_chunks, TC),
        out_shape=jax.ShapeDtypeStruct((B, A), x.dtype),
        in_specs=[
            pl.BlockSpec(memory_space=pltpu.HBM),
            pl.BlockSpec(memory_space=pltpu.VMEM),
            pl.BlockSpec(memory_space=pltpu.VMEM),
            pl.BlockSpec(memory_space=pltpu.VMEM),
            pl.BlockSpec(memory_space=pltpu.VMEM),
        ],
        out_specs=pl.BlockSpec(memory_space=pltpu.HBM),
        scratch_shapes=[
            pltpu.VMEM((_NBUF, TC, S), jnp.float32),
            pltpu.VMEM((_NBUF, TC, A), jnp.float32),
            pltpu.SemaphoreType.DMA((_NBUF,)),
            pltpu.SemaphoreType.DMA((_NBUF,)),
        ],
    )(x, w1, b1, w2, b2)


# speedup vs baseline: 1.2117x; 1.1462x over previous
import jax
import jax.numpy as jnp
from jax.experimental import pallas as pl
from jax.experimental.pallas import tpu as pltpu


def _mlp_body(x_ref, w1_ref, b1_ref, w2_ref, b2_ref, out_ref):
    x = x_ref[...].astype(jnp.bfloat16)
    w1 = w1_ref[...].astype(jnp.bfloat16)
    hid = jnp.dot(x, w1, preferred_element_type=jnp.float32)
    hid = jnp.maximum(hid + b1_ref[...], 0.0).astype(jnp.bfloat16)
    w2 = w2_ref[...].astype(jnp.bfloat16)
    out = jnp.dot(hid, w2, preferred_element_type=jnp.float32)
    out_ref[...] = out + b2_ref[...]


@jax.jit
def kernel(x, w1, b1, w2, b2):
    B, S = x.shape
    H = w1.shape[1]
    A = w2.shape[1]

    TB = min(8192, B)
    nb = pl.cdiv(B, TB)

    return pl.pallas_call(
        _mlp_body,
        out_shape=jax.ShapeDtypeStruct((B, A), x.dtype),
        grid=(nb,),
        in_specs=[
            pl.BlockSpec((TB, S), lambda i: (i, 0)),
            pl.BlockSpec((S, H), lambda i: (0, 0)),
            pl.BlockSpec((1, H), lambda i: (0, 0)),
            pl.BlockSpec((H, A), lambda i: (0, 0)),
            pl.BlockSpec((1, A), lambda i: (0, 0)),
        ],
        out_specs=pl.BlockSpec((TB, A), lambda i: (i, 0)),
        compiler_params=pltpu.CompilerParams(
            dimension_semantics=("parallel",),
        ),
    )(x, w1, b1, w2, b2)
